# baseline (speedup 1.0000x reference)
import jax
import jax.numpy as jnp
from jax.experimental import pallas as pl
from jax.experimental.pallas import tpu as pltpu

_NS = 64
_TK = 8
_RB = 512
_NJ = 4
_NP0 = 16
_NW = 8


def _gelu_exact(v):
    return 0.5 * v * (1.0 + jax.lax.erf(v * 0.7071067811865476))


def _cat_dot(xv, w_s):
    return jnp.concatenate(
        [jnp.dot(xv, w_s[i], preferred_element_type=jnp.float32)
         for i in range(_NW)], axis=1)


def _fused_kernel(x_ref, s0_ref, w1a_ref, b1e_ref, w2_ref, b2_ref, tau_ref,
                  wih_ref, bih_ref, whh_ref, bhh_ref, wq_ref, bq_ref,
                  wk_ref, bk_ref, wv_ref, bv_ref, gamma_ref, beta_ref,
                  wvp_ref, bvp_ref, wo_ref, bo_ref,
                  out_ref,
                  alpha_s, si_s, sw_s, gh_s, wih_s, wv_s, wvp_s, wo_s,
                  svo_s):
    s = pl.program_id(0)
    d = x_ref.shape[2]

    @pl.when(s < _NP0)
    def _phase0():
        b = s // _NJ
        j = s % _NJ
        xb = x_ref[0]
        h = _gelu_exact(jnp.dot(xb, w1a_ref[...],
                                preferred_element_type=jnp.float32)
                        + b1e_ref[...])
        logits = (jax.lax.dot_general(
            w2_ref[...], h, (((0,), (1,)), ((), ())),
            preferred_element_type=jnp.float32) + b2_ref[...]) / (
                jnp.abs(tau_ref[0, 0]) + 0.1)

        slot_ids = jax.lax.broadcasted_iota(jnp.int32, logits.shape, 0)
        neg_inf = jnp.float32(-jnp.inf)
        work = logits
        mx0 = jnp.max(work, axis=0, keepdims=True)
        mx = mx0
        for _ in range(_TK):
            cand = jnp.where(work == mx, slot_ids, _NS)
            sel = cand == jnp.min(cand, axis=0, keepdims=True)
            work = jnp.where(sel, neg_inf, work)
            mx = jnp.max(work, axis=0, keepdims=True)
        e = jnp.where(work == neg_inf, jnp.exp(logits - mx0), 0.0)
        alpha_t = e / jnp.sum(e, axis=0, keepdims=True)
        alpha_s[s] = alpha_t

        contrib_si = jnp.dot(alpha_t, xb,
                             preferred_element_type=jnp.float32)
        ones = jnp.ones((alpha_t.shape[1], sw_s.shape[2]), jnp.float32)
        contrib_sw = jnp.dot(alpha_t, ones,
                             preferred_element_type=jnp.float32)

        @pl.when(j == 0)
        def _():
            si_s[b] = contrib_si
            sw_s[b] = contrib_sw

        @pl.when(j != 0)
        def _():
            si_s[b] += contrib_si
            sw_s[b] += contrib_sw

        idx_even = jnp.minimum(s // 2, _NW - 1)
        idx_odd = jnp.minimum((s + 1) // 2, _NW - 1)

        @pl.when(s % 2 == 0)
        def _():
            wih_s[idx_even] = wih_ref[...]
            wv_s[idx_even] = wv_ref[...]
            wo_s[idx_even] = wo_ref[...]

        @pl.when((s % 2 == 1) | (s == 0))
        def _():
            gh_s[idx_odd] = jnp.dot(
                s0_ref[...], whh_ref[...],
                preferred_element_type=jnp.float32) + bhh_ref[...]
            wvp_s[idx_odd] = wvp_ref[...]

    @pl.when(s >= _NP0)
    def _phase1():
        t = s - _NP0
        bb = t // _NJ

        do_slot = jnp.logical_or(t == 0,
                                 jnp.logical_and(t % _NJ == 1,
                                                 t // _NJ < si_s.shape[0] - 1))
        bnext = jnp.where(t == 0, 0, t // _NJ + 1)

        @pl.when(do_slot)
        def _():
            s0 = s0_ref[...]
            gh = jnp.concatenate([gh_s[i] for i in range(_NW)], axis=1)
            inv = 1.0 / (sw_s[bnext][:, 0:1] + 1e-8)
            xi = si_s[bnext] * inv
            gi = _cat_dot(xi, wih_s) + bih_ref[...]

            r = jax.nn.sigmoid(gi[:, :d] + gh[:, :d])
            z = jax.nn.sigmoid(gi[:, d:2 * d] + gh[:, d:2 * d])
            n = jnp.tanh(gi[:, 2 * d:] + r * gh[:, 2 * d:])
            s_new = (1.0 - z) * n + z * s0

            q = jnp.dot(s_new, wq_ref[...],
                        preferred_element_type=jnp.float32) + bq_ref[...]
            k = jnp.dot(s_new, wk_ref[...],
                        preferred_element_type=jnp.float32) + bk_ref[...]
            v = _cat_dot(s_new, wv_s) + bv_ref[...]
            scale = (d // 4) ** (-0.5)
            att = jax.lax.dot_general(
                q, k, (((1,), (1,)), ((), ())),
                preferred_element_type=jnp.float32) * scale
            att = att - jnp.max(att, axis=1, keepdims=True)
            p = jnp.exp(att)
            p = p / jnp.sum(p, axis=1, keepdims=True)
            s_comm = jnp.dot(p, v, preferred_element_type=jnp.float32)

            y = s_new + s_comm
            mu = jnp.mean(y, axis=1, keepdims=True)
            var = jnp.mean((y - mu) ** 2, axis=1, keepdims=True)
            yln = (y - mu) / jnp.sqrt(var + 1e-5) * gamma_ref[...] \
                + beta_ref[...]
            sv = _cat_dot(yln, wvp_s) + bvp_ref[...]
            svo_s[bnext] = _cat_dot(sv, wo_s)

        out_ref[0] = jax.lax.dot_general(
            alpha_s[t], svo_s[bb], (((0,), (0,)), ((), ())),
            preferred_element_type=jnp.float32) + bo_ref[...]


def kernel(x, slot_init, W1, b1, W2, b2, W_ih, W_hh, b_ih, b_hh, Wq, bq,
           Wk, bk, Wv, bv, gamma, beta, Wvp, bvp, Wo, bo, tau):
    B, T, D = x.shape
    NS = slot_init.shape[0]
    H = W1.shape[1]
    DQ = Wq.shape[1]
    SWL = 128
    IHW = (3 * D) // _NW
    VW = D // _NW
    r2 = lambda v: v.reshape(1, -1)
    b1e = r2(b1 + jnp.mean(slot_init, axis=0) @ W1[D:])

    c0 = lambda s: (0, 0)
    even_m = lambda s: (0, jnp.minimum(s // 2, _NW - 1))
    odd_m = lambda s: (0, jnp.minimum((s + 1) // 2, _NW - 1))

    out = pl.pallas_call(
        _fused_kernel,
        grid=(2 * _NP0,),
        in_specs=[
            pl.BlockSpec((1, _RB, D),
                         lambda s: (jnp.minimum(s, _NP0 - 1) // _NJ,
                                    jnp.minimum(s, _NP0 - 1) % _NJ, 0)),
            pl.BlockSpec((NS, D), c0),
            pl.BlockSpec((D, H), c0),
            pl.BlockSpec((1, H), c0),
            pl.BlockSpec((H, NS), c0),
            pl.BlockSpec((NS, 1), c0),
            pl.BlockSpec((1, 1), c0),
            pl.BlockSpec((D, IHW), even_m),
            pl.BlockSpec((1, 3 * D), c0),
            pl.BlockSpec((D, IHW), odd_m),
            pl.BlockSpec((1, IHW), odd_m),
            pl.BlockSpec((D, DQ), c0),
            pl.BlockSpec((1, DQ), c0),
            pl.BlockSpec((D, DQ), c0),
            pl.BlockSpec((1, DQ), c0),
            pl.BlockSpec((D, VW), even_m),
            pl.BlockSpec((1, D), c0),
            pl.BlockSpec((1, D), c0),
            pl.BlockSpec((1, D), c0),
            pl.BlockSpec((D, VW), odd_m),
            pl.BlockSpec((1, D), c0),
            pl.BlockSpec((D, VW), even_m),
            pl.BlockSpec((1, D), c0),
        ],
        out_specs=pl.BlockSpec(
            (1, _RB, D),
            lambda s: (jnp.where(s < _NP0, 0, (s - _NP0) // _NJ),
                       jnp.where(s < _NP0, 0, (s - _NP0) % _NJ), 0)),
        out_shape=jax.ShapeDtypeStruct((B, T, D), jnp.float32),
        scratch_shapes=[
            pltpu.VMEM((_NP0, NS, _RB), jnp.float32),
            pltpu.VMEM((B, NS, D), jnp.float32),
            pltpu.VMEM((B, NS, SWL), jnp.float32),
            pltpu.VMEM((_NW, NS, IHW), jnp.float32),
            pltpu.VMEM((_NW, D, IHW), jnp.float32),
            pltpu.VMEM((_NW, D, VW), jnp.float32),
            pltpu.VMEM((_NW, D, VW), jnp.float32),
            pltpu.VMEM((_NW, D, VW), jnp.float32),
            pltpu.VMEM((B, NS, D), jnp.float32),
        ],
    )(x, slot_init, W1[:D], b1e, W2, b2.reshape(NS, 1), tau.reshape(1, 1),
      W_ih, r2(b_ih), W_hh, r2(b_hh), Wq, r2(bq), Wk, r2(bk),
      Wv, r2(bv), r2(gamma), r2(beta), Wvp, r2(bvp), Wo, r2(bo))
    return out


# speedup vs baseline: 1.1935x; 1.1935x over previous
import jax
import jax.numpy as jnp
from jax.experimental import pallas as pl
from jax.experimental.pallas import tpu as pltpu

_NS = 64
_TK = 8


def _gelu_exact(v):
    return 0.5 * v * (1.0 + jax.lax.erf(v * 0.7071067811865476))


def _routing_kernel(x_ref, s0_ref, w1a_ref, w1b_ref, b1_ref, w2_ref, b2_ref,
                    tau_ref, whh_ref, bhh_ref,
                    alpha_ref, si_ref, sw_ref, gh_ref):
    j = pl.program_id(1)
    xb = x_ref[0]
    sm = jnp.mean(s0_ref[...], axis=0, keepdims=True)
    b_eff = b1_ref[...] + jnp.dot(sm, w1b_ref[...],
                                  preferred_element_type=jnp.float32)
    h = _gelu_exact(jnp.dot(xb, w1a_ref[...],
                            preferred_element_type=jnp.float32) + b_eff)
    logits = (jax.lax.dot_general(
        w2_ref[...], h, (((0,), (1,)), ((), ())),
        preferred_element_type=jnp.float32) + b2_ref[...]) / (
            jnp.abs(tau_ref[0, 0]) + 0.1)

    gh_ref[...] = jnp.dot(s0_ref[...], whh_ref[...],
                          preferred_element_type=jnp.float32) + bhh_ref[...]

    slot_ids = jax.lax.broadcasted_iota(jnp.int32, logits.shape, 0)
    neg_inf = jnp.float32(-jnp.inf)
    work = logits
    mx0 = jnp.max(work, axis=0, keepdims=True)
    mx = mx0
    for _ in range(_TK):
        cand = jnp.where(work == mx, slot_ids, _NS)
        sel = cand == jnp.min(cand, axis=0, keepdims=True)
        work = jnp.where(sel, neg_inf, work)
        mx = jnp.max(work, axis=0, keepdims=True)
    e = jnp.where(work == neg_inf, jnp.exp(logits - mx0), 0.0)
    alpha_t = e / jnp.sum(e, axis=0, keepdims=True)
    alpha_ref[0] = alpha_t

    contrib_si = jnp.dot(alpha_t, xb,
                         preferred_element_type=jnp.float32)
    ones = jnp.ones((alpha_t.shape[1], sw_ref.shape[2]), jnp.float32)
    contrib_sw = jnp.dot(alpha_t, ones,
                         preferred_element_type=jnp.float32)

    @pl.when(j == 0)
    def _():
        si_ref[0] = contrib_si
        sw_ref[0] = contrib_sw

    @pl.when(j != 0)
    def _():
        si_ref[0] += contrib_si
        sw_ref[0] += contrib_sw


def _slotout_kernel(si_ref, sw_ref, s0_ref, gh_ref, wih_ref, bih_ref,
                    wq_ref, bq_ref, wk_ref, bk_ref, wv_ref, bv_ref,
                    gamma_ref, beta_ref, wvp_ref, bvp_ref, wo_ref,
                    alpha_ref, bo_ref, out_ref, svo_ref):
    b = pl.program_id(0)
    b_sz, ns, d = si_ref.shape

    @pl.when(b == 0)
    def _():
        s0 = s0_ref[...]
        gh = gh_ref[...]
        ghb = jnp.concatenate([gh] * b_sz, axis=0)
        hh = jnp.concatenate([s0] * b_sz, axis=0)

        inv = 1.0 / (sw_ref[:, :, 0:1] + 1e-8)
        xi = (si_ref[...] * inv).reshape(b_sz * ns, d)
        gi = jnp.dot(xi, wih_ref[...],
                     preferred_element_type=jnp.float32) + bih_ref[...]

        r = jax.nn.sigmoid(gi[:, :d] + ghb[:, :d])
        z = jax.nn.sigmoid(gi[:, d:2 * d] + ghb[:, d:2 * d])
        n = jnp.tanh(gi[:, 2 * d:] + r * ghb[:, 2 * d:])
        s_new = (1.0 - z) * n + z * hh

        q = jnp.dot(s_new, wq_ref[...],
                    preferred_element_type=jnp.float32) + bq_ref[...]
        k = jnp.dot(s_new, wk_ref[...],
                    preferred_element_type=jnp.float32) + bk_ref[...]
        v = jnp.dot(s_new, wv_ref[...],
                    preferred_element_type=jnp.float32) + bv_ref[...]
        scale = (d // 4) ** (-0.5)
        comm = []
        for bb in range(b_sz):
            qb = q[bb * ns:(bb + 1) * ns]
            kb = k[bb * ns:(bb + 1) * ns]
            vb = v[bb * ns:(bb + 1) * ns]
            att = jax.lax.dot_general(
                qb, kb, (((1,), (1,)), ((), ())),
                preferred_element_type=jnp.float32) * scale
            att = att - jnp.max(att, axis=1, keepdims=True)
            p = jnp.exp(att)
            p = p / jnp.sum(p, axis=1, keepdims=True)
            comm.append(jnp.dot(p, vb, preferred_element_type=jnp.float32))
        s_comm = jnp.concatenate(comm, axis=0)

        y = s_new + s_comm
        mu = jnp.mean(y, axis=1, keepdims=True)
        var = jnp.mean((y - mu) ** 2, axis=1, keepdims=True)
        yln = (y - mu) / jnp.sqrt(var + 1e-5) * gamma_ref[...] + beta_ref[...]
        sv = jnp.dot(yln, wvp_ref[...],
                     preferred_element_type=jnp.float32) + bvp_ref[...]
        svo_ref[...] = jnp.dot(
            sv, wo_ref[...],
            preferred_element_type=jnp.float32).reshape(b_sz, ns, d)

    out_ref[0] = jax.lax.dot_general(
        alpha_ref[0], svo_ref[b], (((0,), (0,)), ((), ())),
        preferred_element_type=jnp.float32) + bo_ref[...]


def kernel(x, slot_init, W1, b1, W2, b2, W_ih, W_hh, b_ih, b_hh, Wq, bq,
           Wk, bk, Wv, bv, gamma, beta, Wvp, bvp, Wo, bo, tau):
    B, T, D = x.shape
    NS = slot_init.shape[0]
    H = W1.shape[1]
    RB = 2048
    TB = 2048
    SWL = 128
    GHW = (3 * D) // (B * (T // RB))
    W1a, W1b = W1[:D], W1[D:]
    r2 = lambda v: v.reshape(1, -1)

    alpha_t, si, sw, gh = pl.pallas_call(
        _routing_kernel,
        grid=(B, T // RB),
        in_specs=[
            pl.BlockSpec((1, RB, D), lambda b, j: (b, j, 0)),
            pl.BlockSpec((NS, D), lambda b, j: (0, 0)),
            pl.BlockSpec((D, H), lambda b, j: (0, 0)),
            pl.BlockSpec((D, H), lambda b, j: (0, 0)),
            pl.BlockSpec((1, H), lambda b, j: (0, 0)),
            pl.BlockSpec((H, NS), lambda b, j: (0, 0)),
            pl.BlockSpec((NS, 1), lambda b, j: (0, 0)),
            pl.BlockSpec((1, 1), lambda b, j: (0, 0)),
            pl.BlockSpec((D, GHW), lambda b, j, _n=T // RB: (0, b * _n + j)),
            pl.BlockSpec((1, GHW), lambda b, j, _n=T // RB: (0, b * _n + j)),
        ],
        out_specs=[
            pl.BlockSpec((1, NS, RB), lambda b, j: (b, 0, j)),
            pl.BlockSpec((1, NS, D), lambda b, j: (b, 0, 0)),
            pl.BlockSpec((1, NS, SWL), lambda b, j: (b, 0, 0)),
            pl.BlockSpec((NS, GHW), lambda b, j, _n=T // RB: (0, b * _n + j)),
        ],
        out_shape=[
            jax.ShapeDtypeStruct((B, NS, T), jnp.float32),
            jax.ShapeDtypeStruct((B, NS, D), jnp.float32),
            jax.ShapeDtypeStruct((B, NS, SWL), jnp.float32),
            jax.ShapeDtypeStruct((NS, 3 * D), jnp.float32),
        ],
    )(x, slot_init, W1a, W1b, r2(b1), W2, b2.reshape(NS, 1),
      tau.reshape(1, 1), W_hh, r2(b_hh))

    out = pl.pallas_call(
        _slotout_kernel,
        grid=(B,),
        in_specs=[
            pl.BlockSpec((B, NS, D), lambda b: (0, 0, 0)),
            pl.BlockSpec((B, NS, SWL), lambda b: (0, 0, 0)),
            pl.BlockSpec((NS, D), lambda b: (0, 0)),
            pl.BlockSpec((NS, 3 * D), lambda b: (0, 0)),
            pl.BlockSpec((D, 3 * D), lambda b: (0, 0)),
            pl.BlockSpec((1, 3 * D), lambda b: (0, 0)),
            pl.BlockSpec((D, D // 4), lambda b: (0, 0)),
            pl.BlockSpec((1, D // 4), lambda b: (0, 0)),
            pl.BlockSpec((D, D // 4), lambda b: (0, 0)),
            pl.BlockSpec((1, D // 4), lambda b: (0, 0)),
            pl.BlockSpec((D, D), lambda b: (0, 0)),
            pl.BlockSpec((1, D), lambda b: (0, 0)),
            pl.BlockSpec((1, D), lambda b: (0, 0)),
            pl.BlockSpec((1, D), lambda b: (0, 0)),
            pl.BlockSpec((D, D), lambda b: (0, 0)),
            pl.BlockSpec((1, D), lambda b: (0, 0)),
            pl.BlockSpec((D, D), lambda b: (0, 0)),
            pl.BlockSpec((1, NS, T), lambda b: (b, 0, 0)),
            pl.BlockSpec((1, D), lambda b: (0, 0)),
        ],
        out_specs=pl.BlockSpec((1, T, D), lambda b: (b, 0, 0)),
        out_shape=jax.ShapeDtypeStruct((B, T, D), jnp.float32),
        scratch_shapes=[pltpu.VMEM((B, NS, D), jnp.float32)],
    )(si, sw, slot_init, gh, W_ih, r2(b_ih), Wq, r2(bq),
      Wk, r2(bk), Wv, r2(bv), r2(gamma), r2(beta), Wvp, r2(bvp), Wo,
      alpha_t, r2(bo))
    return out
